# Initial kernel scaffold; baseline (speedup 1.0000x reference)
import jax
import jax.numpy as jnp
from jax.experimental import pallas as pl


def kernel(activation, hidden_state, in_proj_w, in_proj_b, out_proj_w, out_proj_b, gru_w_ih, gru_w_hh, gru_b_ih, gru_b_hh, act_w1, act_b1, act_w2, act_b2, ln_g, ln_b, sparsity_k):
    raise NotImplementedError("write your pallas kernel here")


# speedup vs baseline: 3.6096x; 3.6096x over previous
import functools
import math

import jax
import jax.numpy as jnp
from jax import lax
from jax.experimental import pallas as pl
from jax.experimental.pallas import tpu as pltpu
from jax.experimental.pallas import tpu_sc as plsc

N_NEURONS = 131072
D = 256
N_HEADS = 4
K_ACTIVE = 2048
TOP_K = 1024

NC = 2
NS = 16
NW = NC * NS
LANES = 16
CHUNK = N_NEURONS // NW
SLOTS = K_ACTIVE // NW

@functools.cache
def _sc_mesh():
    return plsc.VectorSubcoreMesh(
        core_axis_name="c", subcore_axis_name="s",
        num_cores=NC, num_subcores=NS)


def _worker_id():
    return lax.axis_index("s") * NC + lax.axis_index("c")


def _compact_body(act_hbm, pidx_hbm, pval_hbm, counts_hbm,
                  act_v, idx_buf, val_buf, cnt_v, sem):
    w = _worker_id()
    base = w * CHUNK
    pltpu.async_copy(act_hbm.at[pl.ds(base, CHUNK)], act_v, sem).wait()

    def step(i, cnt):
        v = act_v[pl.ds(i * LANES, LANES)]
        m = v > 0.01
        gi = base + i * LANES + lax.iota(jnp.int32, LANES)
        mi = m.astype(jnp.int32)
        pos = cnt + plsc.cumsum(mi) - 1
        plsc.store_scatter(idx_buf, [pos], gi, mask=m)
        plsc.store_scatter(val_buf, [pos], v, mask=m)
        return cnt + jnp.sum(mi)

    cnt = lax.fori_loop(0, CHUNK // LANES, step, jnp.int32(0))
    cnt_v[...] = jnp.full((LANES,), cnt, jnp.int32)
    pltpu.sync_copy(cnt_v, counts_hbm.at[w])
    pltpu.sync_copy(idx_buf.at[pl.ds(0, CHUNK)], pidx_hbm.at[pl.ds(base, CHUNK)])
    pltpu.sync_copy(val_buf.at[pl.ds(0, CHUNK)], pval_hbm.at[pl.ds(base, CHUNK)])


@functools.cache
def _compact_call():
  return pl.kernel(
    _compact_body,
    out_type=(
        jax.ShapeDtypeStruct((N_NEURONS,), jnp.int32),
        jax.ShapeDtypeStruct((N_NEURONS,), jnp.float32),
        jax.ShapeDtypeStruct((NW, LANES), jnp.int32),
    ),
    mesh=_sc_mesh(),
    compiler_params=pltpu.CompilerParams(needs_layout_passes=False),
    scratch_types=[
        pltpu.VMEM((CHUNK,), jnp.float32),
        pltpu.VMEM((CHUNK + LANES,), jnp.int32),
        pltpu.VMEM((CHUNK + LANES,), jnp.float32),
        pltpu.VMEM((LANES,), jnp.int32),
        pltpu.SemaphoreType.DMA,
    ],
  )


def _gather_body(pidx_hbm, pval_hbm, counts_hbm, hid_hbm,
                 aidx_out, aact_out, astates_out,
                 counts_v, offs_ref, src_ref, aidx_v, aval_v, rows_v, sem):
    w = _worker_id()
    pltpu.async_copy(counts_hbm, counts_v, sem).wait()

    iota = lax.iota(jnp.int32, LANES)
    zeros = jnp.zeros((LANES,), jnp.int32)
    cnt0 = plsc.load_gather(counts_v, [iota, zeros])
    cnt1 = plsc.load_gather(counts_v, [iota + LANES, zeros])
    ex0 = plsc.cumsum(cnt0) - cnt0
    total0 = jnp.sum(cnt0)
    ex1 = plsc.cumsum(cnt1) - cnt1 + total0
    offs_ref[pl.ds(0, LANES)] = ex0
    offs_ref[pl.ds(LANES, LANES)] = ex1

    for q in range(SLOTS // LANES):
        jvec = w * SLOTS + q * LANES + iota
        t = jnp.zeros((LANES,), jnp.int32)
        for u in range(NW):
            s = ex0[u] if u < LANES else ex1[u - LANES]
            t = t + (s <= jvec).astype(jnp.int32)
        t = t - 1
        base_v = plsc.load_gather(offs_ref, [t])
        src_ref[pl.ds(q * LANES, LANES)] = t * CHUNK + (jvec - base_v)

    pltpu.async_copy(pidx_hbm.at[src_ref], aidx_v, sem).wait()
    pltpu.async_copy(pval_hbm.at[src_ref], aval_v, sem).wait()
    pltpu.async_copy(hid_hbm.at[aidx_v], rows_v, sem).wait()

    pltpu.sync_copy(aidx_v, aidx_out.at[pl.ds(w * SLOTS, SLOTS)])
    pltpu.sync_copy(aval_v, aact_out.at[pl.ds(w * SLOTS, SLOTS)])
    pltpu.sync_copy(rows_v, astates_out.at[pl.ds(w * SLOTS, SLOTS)])


@functools.cache
def _gather_call():
  return pl.kernel(
    _gather_body,
    out_type=(
        jax.ShapeDtypeStruct((K_ACTIVE,), jnp.int32),
        jax.ShapeDtypeStruct((K_ACTIVE,), jnp.float32),
        jax.ShapeDtypeStruct((K_ACTIVE, D), jnp.float32),
    ),
    mesh=_sc_mesh(),
    compiler_params=pltpu.CompilerParams(needs_layout_passes=False),
    scratch_types=[
        pltpu.VMEM((NW, LANES), jnp.int32),
        pltpu.VMEM((NW,), jnp.int32),
        pltpu.VMEM((SLOTS,), jnp.int32),
        pltpu.VMEM((SLOTS,), jnp.int32),
        pltpu.VMEM((SLOTS,), jnp.float32),
        pltpu.VMEM((SLOTS, D), jnp.float32),
        pltpu.SemaphoreType.DMA,
    ],
  )


def _dense_kernel(x_ref, aact_ref, ipw_ref, ipb_ref, opw_ref, opb_ref,
                  wih_ref, whh_ref, bih_ref, bhh_ref, w1_ref, b1_ref,
                  w2_ref, b2_ref, lng_ref, lnb_ref,
                  ns_ref, na_ref):
    f32 = jnp.float32
    x = x_ref[...]
    aact = aact_ref[...]
    qkv = lax.dot_general(x, ipw_ref[...], (((1,), (1,)), ((), ())),
                          preferred_element_type=f32) + ipb_ref[...]
    hd = D // N_HEADS
    QB = KB = 1024
    o_heads = []
    for h in range(N_HEADS):
        q = qkv[:, h * hd:(h + 1) * hd]
        k = qkv[:, D + h * hd:D + (h + 1) * hd]
        v = qkv[:, 2 * D + h * hd:2 * D + (h + 1) * hd]
        o_qblocks = []
        for qb in range(K_ACTIVE // QB):
            s = lax.dot_general(q[qb * QB:(qb + 1) * QB], k,
                                (((1,), (1,)), ((), ())),
                                preferred_element_type=f32) / math.sqrt(hd)
            s0 = s[:, 0:KB]
            m0 = jnp.max(s0, axis=-1, keepdims=True)
            e0 = jnp.exp(s0 - m0)
            l0 = jnp.sum(e0, axis=-1, keepdims=True)
            o = jnp.dot(e0, v[0:KB], preferred_element_type=f32) * (1.0 / l0)
            s1 = s[:, KB:2 * KB]
            mb = jnp.max(s1, axis=-1, keepdims=True)
            m1 = jnp.maximum(m0, mb)
            corr = jnp.where(m0 == m1, 0.0, m0 - m1)
            e1 = jnp.exp(s1 - m1)
            lb = jnp.sum(e1, axis=-1, keepdims=True)
            l1 = jnp.exp(corr) * l0 + lb
            o = o * (jnp.exp(corr) * l0) + jnp.dot(
                e1, v[KB:2 * KB], preferred_element_type=f32)
            o_qblocks.append(o * (1.0 / l1))
        o_heads.append(jnp.concatenate(o_qblocks, axis=0))
    o = jnp.concatenate(o_heads, axis=1)
    msg = (lax.dot_general(o, opw_ref[...], (((1,), (1,)), ((), ())),
                           preferred_element_type=f32) + opb_ref[...]) * aact
    gi = lax.dot_general(msg, wih_ref[...], (((1,), (1,)), ((), ())),
                         preferred_element_type=f32) + bih_ref[...]
    gh = lax.dot_general(x, whh_ref[...], (((1,), (1,)), ((), ())),
                         preferred_element_type=f32) + bhh_ref[...]
    r = jax.nn.sigmoid(gi[:, 0:D] + gh[:, 0:D])
    z = jax.nn.sigmoid(gi[:, D:2 * D] + gh[:, D:2 * D])
    n = jnp.tanh(gi[:, 2 * D:3 * D] + r * gh[:, 2 * D:3 * D])
    ns = (1.0 - z) * n + z * x
    mu = jnp.mean(ns, axis=1, keepdims=True)
    var = jnp.mean((ns - mu) ** 2, axis=1, keepdims=True)
    ns = (ns - mu) / jnp.sqrt(var + 1e-5) * lng_ref[...] + lnb_ref[...]

    combined = jnp.concatenate([x, ns], axis=1)
    h1p = lax.dot_general(combined, w1_ref[...], (((1,), (1,)), ((), ())),
                          preferred_element_type=f32) + b1_ref[...]
    h1 = h1p * 0.5 * (1.0 + lax.erf(h1p / math.sqrt(2.0)))
    dpre = lax.dot_general(h1, w2_ref[...], (((1,), (1,)), ((), ())),
                           preferred_element_type=f32)
    delta = jax.nn.sigmoid(dpre[:, 0:1] + b2_ref[0, 0])
    na = jnp.clip(0.7 * aact + 0.3 * delta, 0.0, 1.0)

    na_row = jnp.transpose(na)
    rank = jnp.zeros((K_ACTIVE, 1), f32)
    CB = 256
    col0 = lax.broadcasted_iota(jnp.int32, (1, CB), 1)
    row_i = lax.broadcasted_iota(jnp.int32, (K_ACTIVE, 1), 0)
    for c in range(K_ACTIVE // CB):
        blk = na_row[:, c * CB:(c + 1) * CB]
        gt = (blk > na).astype(f32)
        eq = (blk == na) & ((col0 + c * CB) < row_i)
        rank = rank + jnp.sum(gt + eq.astype(f32), axis=1, keepdims=True)
    sel = rank < float(TOP_K)
    ns_ref[...] = jnp.where(sel, ns, 0.0)
    na_ref[...] = jnp.where(sel, na, 0.0)


def _dense_call(astates, aact, ipw, ipb, opw, opb, wih, whh, bih, bhh,
                w1, b1, w2, b2, lng, lnb):
    return pl.pallas_call(
        _dense_kernel,
        out_shape=(
            jax.ShapeDtypeStruct((K_ACTIVE, D), jnp.float32),
            jax.ShapeDtypeStruct((K_ACTIVE, 1), jnp.float32),
        ),
        compiler_params=pltpu.CompilerParams(
            vmem_limit_bytes=100 * 1024 * 1024),
    )(astates, aact.reshape(K_ACTIVE, 1), ipw, ipb.reshape(1, 3 * D),
      opw, opb.reshape(1, D), wih, whh, bih.reshape(1, 3 * D),
      bhh.reshape(1, 3 * D), w1, b1.reshape(1, D),
      jnp.concatenate([w2, jnp.zeros((127, D), jnp.float32)], axis=0),
      b2.reshape(1, 1), lng.reshape(1, D), lnb.reshape(1, D))


BLK = 1024
N_BLOCKS = N_NEURONS // BLK


def _scatter_kernel(aidx_s, aidx_v_ref, ns_ref, na_ref, hid_ref, act_ref):
    b = pl.program_id(0)
    hid_ref[...] = jnp.zeros_like(hid_ref)
    act_ref[...] = jnp.zeros_like(act_ref)
    av = aidx_v_ref[...]
    start = jnp.sum((av < b * BLK).astype(jnp.int32))
    end = jnp.sum((av < (b + 1) * BLK).astype(jnp.int32))

    def body(j, _):
        g = aidx_s[j]
        loc = g - b * BLK
        hid_ref[pl.ds(loc, 1), :] = ns_ref[pl.ds(j, 1), :]
        act_ref[pl.ds(loc, 1), :] = na_ref[pl.ds(j, 1), :]
        return 0

    lax.fori_loop(start, end, body, 0)


def _scatter_call(aidx, ns_masked, na_masked):
    grid_spec = pltpu.PrefetchScalarGridSpec(
        num_scalar_prefetch=1,
        grid=(N_BLOCKS,),
        in_specs=[
            pl.BlockSpec((LANES, K_ACTIVE // LANES), lambda b, s: (0, 0)),
            pl.BlockSpec((K_ACTIVE, D), lambda b, s: (0, 0)),
            pl.BlockSpec((K_ACTIVE, 1), lambda b, s: (0, 0)),
        ],
        out_specs=[
            pl.BlockSpec((BLK, D), lambda b, s: (b, 0)),
            pl.BlockSpec((BLK, 1), lambda b, s: (b, 0)),
        ],
    )
    hid, act = pl.pallas_call(
        _scatter_kernel,
        grid_spec=grid_spec,
        out_shape=(
            jax.ShapeDtypeStruct((N_NEURONS, D), jnp.float32),
            jax.ShapeDtypeStruct((N_NEURONS, 1), jnp.float32),
        ),
    )(aidx, aidx.reshape(LANES, K_ACTIVE // LANES), ns_masked, na_masked)
    return act.reshape(N_NEURONS), hid


def kernel(activation, hidden_state, in_proj_w, in_proj_b, out_proj_w,
           out_proj_b, gru_w_ih, gru_w_hh, gru_b_ih, gru_b_hh, act_w1,
           act_b1, act_w2, act_b2, ln_g, ln_b, sparsity_k):
    pidx, pval, counts = _compact_call()(activation)
    aidx, aact, astates = _gather_call()(pidx, pval, counts, hidden_state)
    ns_masked, na_masked = _dense_call(
        astates, aact, in_proj_w, in_proj_b, out_proj_w, out_proj_b,
        gru_w_ih, gru_w_hh, gru_b_ih, gru_b_hh, act_w1, act_b1, act_w2,
        act_b2, ln_g, ln_b)
    act_out, hid_out = _scatter_call(aidx, ns_masked, na_masked)
    return (act_out, hid_out)
